# baseline (speedup 1.0000x reference)
import functools

import jax
import jax.numpy as jnp
from jax import lax
from jax.experimental import pallas as pl
from jax.experimental.pallas import tpu as pltpu
from jax.experimental.pallas import tpu_sc as plsc

N = 640000
V = 10000
NC, NS, L = 2, 16, 16
NW = NC * NS
VPW = 320
VPAD = NW * VPW
SLICE = 336
SPAD = (NW - 1) * VPW + SLICE
CH = 128
NPAD = N + 144
BLK = 1024


def _relu(x):
    return jnp.maximum(x, 0.0)


def _dot(a, b):
    return jnp.dot(a, b, preferred_element_type=jnp.float32)


def _bf(x):
    return x.astype(jnp.bfloat16)


def _mlp2_body(x_ref, w1_ref, b1_ref, w2_ref, b2_ref, o_ref):
    h = _relu(_dot(_bf(x_ref[...]), w1_ref[...]) + b1_ref[...])
    o_ref[...] = _relu(_dot(_bf(h), w2_ref[...]) + b2_ref[...])


def _voxg_body(x_ref, wv_ref, bv_ref, w3l_ref, o_ref):
    h = _relu(_dot(_bf(x_ref[...]), wv_ref[...]) + bv_ref[...])
    o_ref[...] = _dot(_bf(h), w3l_ref[...])


def _mid_body(pf2_ref, pgf_ref, w3r_ref, b3_ref, w4_ref, b4_ref, o_ref):
    pf4 = _relu(pgf_ref[...].astype(jnp.float32)
                + _dot(pf2_ref[...], w3r_ref[...]) + b3_ref[...])
    o_ref[...] = _relu(_dot(_bf(pf4), w4_ref[...]) + b4_ref[...])


def _vout_body(x_ref, wv2_ref, bv2_ref, o_ref):
    o_ref[...] = _relu(_dot(_bf(x_ref[...]), wv2_ref[...]) + bv2_ref[...])


def _full(shape):
    return pl.BlockSpec(shape, lambda i: (0, 0))


def _sc_mesh():
    return plsc.VectorSubcoreMesh(
        core_axis_name="c", subcore_axis_name="s",
        num_cores=NC, num_subcores=NS)


def _wid():
    return lax.axis_index("s") * NC + lax.axis_index("c")


def _sread(vec_ref, i):
    return vec_ref[pl.ds(i, L)][0]


def _make_segmax(C):
    VW = 16
    GN = C // VW
    ALIGN = 1
    CHB = CH + ALIGN
    CHW = CHB * C
    IBW = CH + 24

    def body(data_hbm, ids_hbm, starts_hbm, out_hbm,
             starts_v, buf, idb, res, acc, sem0, sem1):
        wid = _wid()
        vlo = wid * VPW
        nv = jnp.minimum(VPW, V - vlo)
        pltpu.sync_copy(starts_hbm.at[pl.ds(vlo, SLICE)], starts_v)

        zeros = tuple(jnp.zeros((VW,), jnp.float32) for _ in range(GN))

        def zero_body(i, c):
            for g in range(GN):
                res[pl.ds(i * C + g * VW, VW)] = zeros[g]
            return c

        lax.fori_loop(0, VPW, zero_body, 0)
        for g in range(GN):
            acc[pl.ds(g * VW, VW)] = zeros[g]

        r0 = _sread(starts_v, 0)
        r1 = _sread(starts_v, nv)
        nch = (r1 - r0 + (CH - 1)) // CH

        def issue(k, b, sem):
            cb = r0 + k * CH
            cba = (cb // ALIGN) * ALIGN
            pltpu.async_copy(
                data_hbm.at[pl.ds(pl.multiple_of(cba * C, C), CHW)],
                buf.at[pl.ds(b * CHW, CHW)], sem)
            pltpu.async_copy(
                ids_hbm.at[pl.ds((cb // 8) * 8, IBW)],
                idb.at[pl.ds(b * IBW, IBW)], sem)

        @pl.when(nch > 0)
        def _():
            issue(0, 0, sem0)

        @pl.when(nch > 1)
        def _():
            issue(1, 1, sem1)

        def step(k, b, sem):
            cb = r0 + k * CH
            boff = b * CHW + (cb - (cb // ALIGN) * ALIGN) * C

            @pl.when(k < nch)
            def _():
                pltpu.make_async_copy(
                    data_hbm.at[pl.ds(0, CHW)],
                    buf.at[pl.ds(b * CHW, CHW)], sem).wait()
                pltpu.make_async_copy(
                    ids_hbm.at[pl.ds(0, IBW)],
                    idb.at[pl.ds(b * IBW, IBW)], sem).wait()

            ce = jnp.minimum(cb + CH, r1)
            rem = jnp.maximum(ce - cb, 0)
            ioff = b * IBW + (cb - (cb // 8) * 8)

            def row_step(r, id_r, accs):
                id_n = idb[pl.ds(ioff + r + 1, L)][0]
                cur = tuple(
                    jnp.maximum(accs[g], buf[pl.ds(boff + r * C + g * VW, VW)])
                    for g in range(GN))
                done = id_n != id_r

                @pl.when(done)
                def _():
                    for g in range(GN):
                        res[pl.ds((id_r - vlo) * C + g * VW, VW)] = cur[g]

                keep = (~done).astype(jnp.float32)
                return id_n, tuple(cur[g] * keep for g in range(GN))

            def rb2(t, st):
                id_r, accs = st
                id_r, accs = row_step(2 * t, id_r, accs)
                id_r, accs = row_step(2 * t + 1, id_r, accs)
                return id_r, accs

            def rb1(r, st):
                id_r, accs = st
                return row_step(r, id_r, accs)

            st = (idb[pl.ds(ioff, L)][0],
                  tuple(acc[pl.ds(g * VW, VW)] for g in range(GN)))
            st = lax.fori_loop(0, rem // 2, rb2, st)
            st = lax.fori_loop(2 * (rem // 2), rem, rb1, st)
            for g in range(GN):
                acc[pl.ds(g * VW, VW)] = st[1][g]

            @pl.when(k + 2 < nch)
            def _():
                issue(k + 2, b, sem)

        def pair(t, c):
            step(2 * t, 0, sem0)
            step(2 * t + 1, 1, sem1)
            return c

        lax.fori_loop(0, (nch + 1) // 2, pair, 0)
        pltpu.sync_copy(
            res, out_hbm.at[pl.ds(pl.multiple_of(vlo * C, C), VPW * C)])

    return pl.kernel(
        body,
        out_type=jax.ShapeDtypeStruct((VPAD * C,), jnp.float32),
        mesh=_sc_mesh(),
        scratch_types=[
            pltpu.VMEM((SLICE,), jnp.int32),
            pltpu.VMEM((2 * CHW,), jnp.float32),
            pltpu.VMEM((2 * IBW,), jnp.int32),
            pltpu.VMEM((VPW * C,), jnp.float32),
            pltpu.VMEM((C,), jnp.float32),
            pltpu.SemaphoreType.DMA,
            pltpu.SemaphoreType.DMA,
        ],
    )


_G = 80
_RPW = N // NW
_BB = 400
_NG = _BB // _G
_NBB = _RPW // _BB


def _make_gather():
    def body(table_hbm, ids_hbm, out_hbm, idx_v, rows_v,
             sg0, sg1, ss0, ss1):
        r0w = _wid() * _RPW
        pltpu.sync_copy(ids_hbm.at[pl.ds(r0w, _RPW)], idx_v)

        def issue_gathers(k, b, sem):
            for g in range(_NG):
                pltpu.async_copy(
                    table_hbm.at[idx_v.at[pl.ds(k * _BB + g * _G, _G)]],
                    rows_v.at[b].at[pl.ds(g * _G, _G)], sem)

        issue_gathers(0, 0, sg0)
        issue_gathers(1, 1, sg1)

        def store_desc(k, b, sem):
            return pltpu.make_async_copy(
                rows_v.at[b], out_hbm.at[pl.ds(r0w + k * _BB, _BB)], sem)

        def step(k, b, sg, ss):
            pltpu.make_async_copy(
                out_hbm.at[pl.ds(r0w, _BB)], rows_v.at[b], sg).wait()
            store_desc(k, b, ss).start()

            @pl.when(k + 2 < _NBB)
            def _():
                store_desc(k, b, ss).wait()
                issue_gathers(k + 2, b, sg)

        def pair(t, c):
            step(2 * t, 0, sg0, ss0)
            step(2 * t + 1, 1, sg1, ss1)
            return c

        lax.fori_loop(0, _NBB // 2, pair, 0)
        store_desc(_NBB - 2, 0, ss0).wait()
        store_desc(_NBB - 1, 1, ss1).wait()

    return pl.kernel(
        body,
        out_type=jax.ShapeDtypeStruct((N, 128), jnp.float32),
        mesh=_sc_mesh(),
        scratch_types=[
            pltpu.VMEM((_RPW,), jnp.int32),
            pltpu.VMEM((2, _BB, 128), jnp.float32),
            pltpu.SemaphoreType.DMA,
            pltpu.SemaphoreType.DMA,
            pltpu.SemaphoreType.DMA,
            pltpu.SemaphoreType.DMA,
        ],
    )


def kernel(inp_feat, W1, b1, W2, b2, Wv1, bv1, W3, b3, W4, b4, Wv2, bv2,
           vox2point_idx):
    ids = vox2point_idx.astype(jnp.int32)
    hist = jnp.zeros((V,), jnp.int32).at[ids].add(1)
    starts = jnp.concatenate(
        [jnp.zeros((1,), jnp.int32), jnp.cumsum(hist, dtype=jnp.int32),
         jnp.full((SPAD - (V + 1),), N, jnp.int32)])
    ids_pad = jnp.concatenate([ids, jnp.full((CH + 32,), V, jnp.int32)])

    pf2 = pl.pallas_call(
        _mlp2_body,
        grid=(N // BLK,),
        in_specs=[
            pl.BlockSpec((BLK, 6), lambda i: (i, 0)),
            _full((6, 64)), _full((1, 64)), _full((64, 64)), _full((1, 64)),
        ],
        out_specs=pl.BlockSpec((BLK, 64), lambda i: (i, 0)),
        out_shape=jax.ShapeDtypeStruct((NPAD, 64), jnp.float32),
    )(inp_feat, W1.T.astype(jnp.bfloat16), b1.reshape(1, 64),
      W2.T.astype(jnp.bfloat16), b2.reshape(1, 64))

    vox1 = _make_segmax(64)(pf2.reshape(-1), ids_pad, starts).reshape(VPAD, 64)

    voxg = pl.pallas_call(
        _voxg_body,
        grid=(VPAD // BLK,),
        in_specs=[
            pl.BlockSpec((BLK, 64), lambda i: (i, 0)),
            _full((64, 64)), _full((1, 64)), _full((64, 128)),
        ],
        out_specs=pl.BlockSpec((BLK, 128), lambda i: (i, 0)),
        out_shape=jax.ShapeDtypeStruct((VPAD, 128), jnp.float32),
    )(vox1, Wv1.T.astype(jnp.bfloat16), bv1.reshape(1, 64),
      W3[:, :64].T.astype(jnp.bfloat16))

    pgf = _make_gather()(voxg, ids)

    pf5 = pl.pallas_call(
        _mid_body,
        grid=(N // BLK,),
        in_specs=[
            pl.BlockSpec((BLK, 64), lambda i: (i, 0)),
            pl.BlockSpec((BLK, 128), lambda i: (i, 0)),
            _full((64, 128)), _full((1, 128)), _full((128, 128)),
            _full((1, 128)),
        ],
        out_specs=pl.BlockSpec((BLK, 128), lambda i: (i, 0)),
        out_shape=jax.ShapeDtypeStruct((NPAD, 128), jnp.float32),
    )(pf2, pgf, W3[:, 64:].T.astype(jnp.bfloat16), b3.reshape(1, 128),
      W4.T.astype(jnp.bfloat16), b4.reshape(1, 128))

    vox2 = _make_segmax(128)(pf5.reshape(-1), ids_pad, starts).reshape(VPAD, 128)

    out = pl.pallas_call(
        _vout_body,
        grid=(10,),
        in_specs=[
            pl.BlockSpec((V // 10, 128), lambda i: (i, 0)),
            _full((128, 128)), _full((1, 128)),
        ],
        out_specs=pl.BlockSpec((V // 10, 128), lambda i: (i, 0)),
        out_shape=jax.ShapeDtypeStruct((V, 128), jnp.float32),
    )(vox2, Wv2.T.astype(jnp.bfloat16), bv2.reshape(1, 128))

    return out


# speedup vs baseline: 1.1694x; 1.1694x over previous
import functools

import jax
import jax.numpy as jnp
from jax import lax
from jax.experimental import pallas as pl
from jax.experimental.pallas import tpu as pltpu
from jax.experimental.pallas import tpu_sc as plsc

N = 640000
V = 10000
NC, NS, L = 2, 16, 16
NW = NC * NS
VPW = 320
VPAD = NW * VPW
SLICE = 336
SPAD = (NW - 1) * VPW + SLICE
CH = 128
NPAD = N + 144
BLK = 2560


def _relu(x):
    return jnp.maximum(x, 0.0)


def _dot(a, b):
    return jnp.dot(a, b, preferred_element_type=jnp.float32)


def _bf(x):
    return x.astype(jnp.bfloat16)


def _mlp2_body(x_ref, w1_ref, b1_ref, w2_ref, b2_ref, o_ref):
    h = _relu(_dot(_bf(x_ref[...]), w1_ref[...]) + b1_ref[...])
    o_ref[...] = _relu(_dot(_bf(h), w2_ref[...]) + b2_ref[...])


def _voxg_body(x_ref, wv_ref, bv_ref, w3l_ref, o_ref):
    h = _relu(_dot(_bf(x_ref[...]), wv_ref[...]) + bv_ref[...])
    o_ref[...] = _dot(_bf(h), w3l_ref[...])


def _mid_body(pf2_ref, pgf_ref, w3r_ref, b3_ref, w4_ref, b4_ref, o_ref):
    pf4 = _relu(pgf_ref[...].astype(jnp.float32)
                + _dot(_bf(pf2_ref[...]), w3r_ref[...]) + b3_ref[...])
    o_ref[...] = _relu(_dot(_bf(pf4), w4_ref[...]) + b4_ref[...])


def _vout_body(x_ref, wv2_ref, bv2_ref, o_ref):
    o_ref[...] = _relu(_dot(_bf(x_ref[...]), wv2_ref[...]) + bv2_ref[...])


def _full(shape):
    return pl.BlockSpec(shape, lambda i: (0, 0))


def _sc_mesh():
    return plsc.VectorSubcoreMesh(
        core_axis_name="c", subcore_axis_name="s",
        num_cores=NC, num_subcores=NS)


def _wid():
    return lax.axis_index("s") * NC + lax.axis_index("c")


def _sread(vec_ref, i):
    return vec_ref[pl.ds(i, L)][0]


def _make_segmax(C):
    VW = 16
    GN = C // VW
    ALIGN = 1
    CHB = CH + ALIGN
    CHW = CHB * C
    IBW = CH + 24

    def body(data_hbm, ids_hbm, starts_hbm, out_hbm,
             starts_v, buf, idb, res, acc, sem0, sem1):
        wid = _wid()
        vlo = wid * VPW
        nv = jnp.minimum(VPW, V - vlo)
        pltpu.sync_copy(starts_hbm.at[pl.ds(vlo, SLICE)], starts_v)

        zeros = tuple(jnp.zeros((VW,), jnp.float32) for _ in range(GN))

        def zero_body(i, c):
            for g in range(GN):
                res[pl.ds(i * C + g * VW, VW)] = zeros[g]
            return c

        lax.fori_loop(0, VPW, zero_body, 0)
        for g in range(GN):
            acc[pl.ds(g * VW, VW)] = zeros[g]

        r0 = _sread(starts_v, 0)
        r1 = _sread(starts_v, nv)
        nch = (r1 - r0 + (CH - 1)) // CH

        def issue(k, b, sem):
            cb = r0 + k * CH
            cba = (cb // ALIGN) * ALIGN
            pltpu.async_copy(
                data_hbm.at[pl.ds(pl.multiple_of(cba * C, C), CHW)],
                buf.at[pl.ds(b * CHW, CHW)], sem)
            pltpu.async_copy(
                ids_hbm.at[pl.ds((cb // 8) * 8, IBW)],
                idb.at[pl.ds(b * IBW, IBW)], sem)

        @pl.when(nch > 0)
        def _():
            issue(0, 0, sem0)

        @pl.when(nch > 1)
        def _():
            issue(1, 1, sem1)

        def step(k, b, sem):
            cb = r0 + k * CH
            boff = b * CHW + (cb - (cb // ALIGN) * ALIGN) * C

            @pl.when(k < nch)
            def _():
                pltpu.make_async_copy(
                    data_hbm.at[pl.ds(0, CHW)],
                    buf.at[pl.ds(b * CHW, CHW)], sem).wait()
                pltpu.make_async_copy(
                    ids_hbm.at[pl.ds(0, IBW)],
                    idb.at[pl.ds(b * IBW, IBW)], sem).wait()

            ce = jnp.minimum(cb + CH, r1)
            rem = jnp.maximum(ce - cb, 0)
            ioff = b * IBW + (cb - (cb // 8) * 8)

            def row_step(r, id_r, accs):
                id_n = idb[pl.ds(ioff + r + 1, L)][0]
                cur = tuple(
                    jnp.maximum(accs[g], buf[pl.ds(boff + r * C + g * VW, VW)])
                    for g in range(GN))
                done = id_n != id_r

                @pl.when(done)
                def _():
                    for g in range(GN):
                        res[pl.ds((id_r - vlo) * C + g * VW, VW)] = cur[g]

                keep = (~done).astype(jnp.float32)
                return id_n, tuple(cur[g] * keep for g in range(GN))

            def rb2(t, st):
                id_r, accs = st
                id_r, accs = row_step(2 * t, id_r, accs)
                id_r, accs = row_step(2 * t + 1, id_r, accs)
                return id_r, accs

            def rb1(r, st):
                id_r, accs = st
                return row_step(r, id_r, accs)

            st = (idb[pl.ds(ioff, L)][0],
                  tuple(acc[pl.ds(g * VW, VW)] for g in range(GN)))
            st = lax.fori_loop(0, rem // 2, rb2, st)
            st = lax.fori_loop(2 * (rem // 2), rem, rb1, st)
            for g in range(GN):
                acc[pl.ds(g * VW, VW)] = st[1][g]

            @pl.when(k + 2 < nch)
            def _():
                issue(k + 2, b, sem)

        def pair(t, c):
            step(2 * t, 0, sem0)
            step(2 * t + 1, 1, sem1)
            return c

        lax.fori_loop(0, (nch + 1) // 2, pair, 0)
        pltpu.sync_copy(
            res, out_hbm.at[pl.ds(pl.multiple_of(vlo * C, C), VPW * C)])

    return pl.kernel(
        body,
        out_type=jax.ShapeDtypeStruct((VPAD * C,), jnp.float32),
        mesh=_sc_mesh(),
        scratch_types=[
            pltpu.VMEM((SLICE,), jnp.int32),
            pltpu.VMEM((2 * CHW,), jnp.float32),
            pltpu.VMEM((2 * IBW,), jnp.int32),
            pltpu.VMEM((VPW * C,), jnp.float32),
            pltpu.VMEM((C,), jnp.float32),
            pltpu.SemaphoreType.DMA,
            pltpu.SemaphoreType.DMA,
        ],
    )


_G = 80
_RPW = N // NW
_BB = 400
_NG = _BB // _G
_NBB = _RPW // _BB


def _make_gather():
    def body(table_hbm, ids_hbm, out_hbm, idx_v, rows_v,
             sg0, sg1, ss0, ss1):
        r0w = _wid() * _RPW
        pltpu.sync_copy(ids_hbm.at[pl.ds(r0w, _RPW)], idx_v)

        def issue_gathers(k, b, sem):
            for g in range(_NG):
                pltpu.async_copy(
                    table_hbm.at[idx_v.at[pl.ds(k * _BB + g * _G, _G)]],
                    rows_v.at[b].at[pl.ds(g * _G, _G)], sem)

        issue_gathers(0, 0, sg0)
        issue_gathers(1, 1, sg1)

        def store_desc(k, b, sem):
            return pltpu.make_async_copy(
                rows_v.at[b], out_hbm.at[pl.ds(r0w + k * _BB, _BB)], sem)

        def step(k, b, sg, ss):
            pltpu.make_async_copy(
                out_hbm.at[pl.ds(r0w, _BB)], rows_v.at[b], sg).wait()
            store_desc(k, b, ss).start()

            @pl.when(k + 2 < _NBB)
            def _():
                store_desc(k, b, ss).wait()
                issue_gathers(k + 2, b, sg)

        def pair(t, c):
            step(2 * t, 0, sg0, ss0)
            step(2 * t + 1, 1, sg1, ss1)
            return c

        lax.fori_loop(0, _NBB // 2, pair, 0)
        store_desc(_NBB - 2, 0, ss0).wait()
        store_desc(_NBB - 1, 1, ss1).wait()

    return pl.kernel(
        body,
        out_type=jax.ShapeDtypeStruct((N, 128), jnp.float32),
        mesh=_sc_mesh(),
        scratch_types=[
            pltpu.VMEM((_RPW,), jnp.int32),
            pltpu.VMEM((2, _BB, 128), jnp.float32),
            pltpu.SemaphoreType.DMA,
            pltpu.SemaphoreType.DMA,
            pltpu.SemaphoreType.DMA,
            pltpu.SemaphoreType.DMA,
        ],
    )


def kernel(inp_feat, W1, b1, W2, b2, Wv1, bv1, W3, b3, W4, b4, Wv2, bv2,
           vox2point_idx):
    ids = vox2point_idx.astype(jnp.int32)
    hist = jnp.zeros((V,), jnp.int32).at[ids].add(1)
    starts = jnp.concatenate(
        [jnp.zeros((1,), jnp.int32), jnp.cumsum(hist, dtype=jnp.int32),
         jnp.full((SPAD - (V + 1),), N, jnp.int32)])
    ids_pad = jnp.concatenate([ids, jnp.full((CH + 32,), V, jnp.int32)])

    pf2 = pl.pallas_call(
        _mlp2_body,
        grid=(N // BLK,),
        in_specs=[
            pl.BlockSpec((BLK, 6), lambda i: (i, 0)),
            _full((6, 64)), _full((1, 64)), _full((64, 64)), _full((1, 64)),
        ],
        out_specs=pl.BlockSpec((BLK, 64), lambda i: (i, 0)),
        out_shape=jax.ShapeDtypeStruct((NPAD, 64), jnp.float32),
    )(inp_feat, W1.T.astype(jnp.bfloat16), b1.reshape(1, 64),
      W2.T.astype(jnp.bfloat16), b2.reshape(1, 64))

    vox1 = _make_segmax(64)(pf2.reshape(-1), ids_pad, starts).reshape(VPAD, 64)

    voxg = pl.pallas_call(
        _voxg_body,
        grid=(VPAD // BLK,),
        in_specs=[
            pl.BlockSpec((BLK, 64), lambda i: (i, 0)),
            _full((64, 64)), _full((1, 64)), _full((64, 128)),
        ],
        out_specs=pl.BlockSpec((BLK, 128), lambda i: (i, 0)),
        out_shape=jax.ShapeDtypeStruct((VPAD, 128), jnp.float32),
    )(vox1, Wv1.T.astype(jnp.bfloat16), bv1.reshape(1, 64),
      W3[:, :64].T.astype(jnp.bfloat16))

    pgf = _make_gather()(voxg, ids)

    pf5 = pl.pallas_call(
        _mid_body,
        grid=(N // BLK,),
        in_specs=[
            pl.BlockSpec((BLK, 64), lambda i: (i, 0)),
            pl.BlockSpec((BLK, 128), lambda i: (i, 0)),
            _full((64, 128)), _full((1, 128)), _full((128, 128)),
            _full((1, 128)),
        ],
        out_specs=pl.BlockSpec((BLK, 128), lambda i: (i, 0)),
        out_shape=jax.ShapeDtypeStruct((NPAD, 128), jnp.float32),
    )(pf2, pgf, W3[:, 64:].T.astype(jnp.bfloat16), b3.reshape(1, 128),
      W4.T.astype(jnp.bfloat16), b4.reshape(1, 128))

    vox2 = _make_segmax(128)(pf5.reshape(-1), ids_pad, starts).reshape(VPAD, 128)

    out = pl.pallas_call(
        _vout_body,
        grid=(10,),
        in_specs=[
            pl.BlockSpec((V // 10, 128), lambda i: (i, 0)),
            _full((128, 128)), _full((1, 128)),
        ],
        out_specs=pl.BlockSpec((V // 10, 128), lambda i: (i, 0)),
        out_shape=jax.ShapeDtypeStruct((V, 128), jnp.float32),
    )(vox2, Wv2.T.astype(jnp.bfloat16), bv2.reshape(1, 128))

    return out


# speedup vs baseline: 1.2427x; 1.0627x over previous
import functools

import jax
import jax.numpy as jnp
from jax import lax
from jax.experimental import pallas as pl
from jax.experimental.pallas import tpu as pltpu
from jax.experimental.pallas import tpu_sc as plsc

N = 640000
V = 10000
NC, NS, L = 2, 16, 16
NW = NC * NS
VPW = 320
VPAD = NW * VPW
SLICE = 336
SPAD = (NW - 1) * VPW + SLICE
CH = 128
NPAD = N + 144
BLK = 5120


def _relu(x):
    return jnp.maximum(x, 0.0)


def _dot(a, b):
    return jnp.dot(a, b, preferred_element_type=jnp.float32)


def _bf(x):
    return x.astype(jnp.bfloat16)


def _mlp2_body(x_ref, w1_ref, b1_ref, w2_ref, b2_ref, o_ref):
    h = _relu(_dot(_bf(x_ref[...]), w1_ref[...]) + b1_ref[...])
    o_ref[...] = _relu(_dot(_bf(h), w2_ref[...]) + b2_ref[...])


def _voxg_body(x_ref, wv_ref, bv_ref, w3l_ref, o_ref):
    h = _relu(_dot(_bf(x_ref[...]), wv_ref[...]) + bv_ref[...])
    o_ref[...] = _dot(_bf(h), w3l_ref[...])


def _mid_body(pf2_ref, pgf_ref, w3r_ref, b3_ref, w4_ref, b4_ref, o_ref):
    pf4 = _relu(pgf_ref[...].astype(jnp.float32)
                + _dot(_bf(pf2_ref[...]), w3r_ref[...]) + b3_ref[...])
    o_ref[...] = _relu(_dot(_bf(pf4), w4_ref[...]) + b4_ref[...])


def _vout_body(x_ref, wv2_ref, bv2_ref, o_ref):
    o_ref[...] = _relu(_dot(_bf(x_ref[...]), wv2_ref[...]) + bv2_ref[...])


def _full(shape):
    return pl.BlockSpec(shape, lambda i: (0, 0))


def _sc_mesh():
    return plsc.VectorSubcoreMesh(
        core_axis_name="c", subcore_axis_name="s",
        num_cores=NC, num_subcores=NS)


def _wid():
    return lax.axis_index("s") * NC + lax.axis_index("c")


def _sread(vec_ref, i):
    return vec_ref[pl.ds(i, L)][0]


def _make_segmax(C):
    VW = 16
    GN = C // VW
    ALIGN = 1
    CHB = CH + ALIGN
    CHW = CHB * C
    IBW = CH + 24

    def body(data_hbm, ids_hbm, starts_hbm, out_hbm,
             starts_v, buf, idb, res, acc, sem0, sem1):
        wid = _wid()
        vlo = wid * VPW
        nv = jnp.minimum(VPW, V - vlo)
        pltpu.sync_copy(starts_hbm.at[pl.ds(vlo, SLICE)], starts_v)

        zeros = tuple(jnp.zeros((VW,), jnp.float32) for _ in range(GN))

        def zero_body(i, c):
            for g in range(GN):
                res[pl.ds(i * C + g * VW, VW)] = zeros[g]
            return c

        lax.fori_loop(0, VPW, zero_body, 0)
        for g in range(GN):
            acc[pl.ds(g * VW, VW)] = zeros[g]

        r0 = _sread(starts_v, 0)
        r1 = _sread(starts_v, nv)
        nch = (r1 - r0 + (CH - 1)) // CH

        def issue(k, b, sem):
            cb = r0 + k * CH
            cba = (cb // ALIGN) * ALIGN
            pltpu.async_copy(
                data_hbm.at[pl.ds(pl.multiple_of(cba * C, C), CHW)],
                buf.at[pl.ds(b * CHW, CHW)], sem)
            pltpu.async_copy(
                ids_hbm.at[pl.ds((cb // 8) * 8, IBW)],
                idb.at[pl.ds(b * IBW, IBW)], sem)

        @pl.when(nch > 0)
        def _():
            issue(0, 0, sem0)

        @pl.when(nch > 1)
        def _():
            issue(1, 1, sem1)

        def step(k, b, sem):
            cb = r0 + k * CH
            boff = b * CHW + (cb - (cb // ALIGN) * ALIGN) * C

            @pl.when(k < nch)
            def _():
                pltpu.make_async_copy(
                    data_hbm.at[pl.ds(0, CHW)],
                    buf.at[pl.ds(b * CHW, CHW)], sem).wait()
                pltpu.make_async_copy(
                    ids_hbm.at[pl.ds(0, IBW)],
                    idb.at[pl.ds(b * IBW, IBW)], sem).wait()

            ce = jnp.minimum(cb + CH, r1)
            rem = jnp.maximum(ce - cb, 0)
            ioff = b * IBW + (cb - (cb // 8) * 8)

            def row_step(r, id_r, accs):
                id_n = idb[pl.ds(ioff + r + 1, L)][0]
                cur = tuple(
                    jnp.maximum(accs[g], buf[pl.ds(boff + r * C + g * VW, VW)])
                    for g in range(GN))
                done = id_n != id_r

                @pl.when(done)
                def _():
                    for g in range(GN):
                        res[pl.ds((id_r - vlo) * C + g * VW, VW)] = cur[g]

                keep = (~done).astype(jnp.float32)
                return id_n, tuple(cur[g] * keep for g in range(GN))

            def rb2(t, st):
                id_r, accs = st
                id_r, accs = row_step(2 * t, id_r, accs)
                id_r, accs = row_step(2 * t + 1, id_r, accs)
                return id_r, accs

            def rb1(r, st):
                id_r, accs = st
                return row_step(r, id_r, accs)

            st = (idb[pl.ds(ioff, L)][0],
                  tuple(acc[pl.ds(g * VW, VW)] for g in range(GN)))
            st = lax.fori_loop(0, rem // 2, rb2, st)
            st = lax.fori_loop(2 * (rem // 2), rem, rb1, st)
            for g in range(GN):
                acc[pl.ds(g * VW, VW)] = st[1][g]

            @pl.when(k + 2 < nch)
            def _():
                issue(k + 2, b, sem)

        def pair(t, c):
            step(2 * t, 0, sem0)
            step(2 * t + 1, 1, sem1)
            return c

        lax.fori_loop(0, (nch + 1) // 2, pair, 0)
        pltpu.sync_copy(
            res, out_hbm.at[pl.ds(pl.multiple_of(vlo * C, C), VPW * C)])

    return pl.kernel(
        body,
        out_type=jax.ShapeDtypeStruct((VPAD * C,), jnp.float32),
        mesh=_sc_mesh(),
        scratch_types=[
            pltpu.VMEM((SLICE,), jnp.int32),
            pltpu.VMEM((2 * CHW,), jnp.float32),
            pltpu.VMEM((2 * IBW,), jnp.int32),
            pltpu.VMEM((VPW * C,), jnp.float32),
            pltpu.VMEM((C,), jnp.float32),
            pltpu.SemaphoreType.DMA,
            pltpu.SemaphoreType.DMA,
        ],
    )


_G = 80
_RPW = N // NW
_BB = 400
_NG = _BB // _G
_NBB = _RPW // _BB


def _make_gather():
    def body(table_hbm, ids_hbm, out_hbm, idx_v, rows_v,
             sg0, sg1, ss0, ss1):
        r0w = _wid() * _RPW
        pltpu.sync_copy(ids_hbm.at[pl.ds(r0w, _RPW)], idx_v)

        def issue_gathers(k, b, sem):
            for g in range(_NG):
                pltpu.async_copy(
                    table_hbm.at[idx_v.at[pl.ds(k * _BB + g * _G, _G)]],
                    rows_v.at[b].at[pl.ds(g * _G, _G)], sem)

        issue_gathers(0, 0, sg0)
        issue_gathers(1, 1, sg1)

        def store_desc(k, b, sem):
            return pltpu.make_async_copy(
                rows_v.at[b], out_hbm.at[pl.ds(r0w + k * _BB, _BB)], sem)

        def step(k, b, sg, ss):
            pltpu.make_async_copy(
                out_hbm.at[pl.ds(r0w, _BB)], rows_v.at[b], sg).wait()
            store_desc(k, b, ss).start()

            @pl.when(k + 2 < _NBB)
            def _():
                store_desc(k, b, ss).wait()
                issue_gathers(k + 2, b, sg)

        def pair(t, c):
            step(2 * t, 0, sg0, ss0)
            step(2 * t + 1, 1, sg1, ss1)
            return c

        lax.fori_loop(0, _NBB // 2, pair, 0)
        store_desc(_NBB - 2, 0, ss0).wait()
        store_desc(_NBB - 1, 1, ss1).wait()

    return pl.kernel(
        body,
        out_type=jax.ShapeDtypeStruct((N, 128), jnp.float32),
        mesh=_sc_mesh(),
        scratch_types=[
            pltpu.VMEM((_RPW,), jnp.int32),
            pltpu.VMEM((2, _BB, 128), jnp.float32),
            pltpu.SemaphoreType.DMA,
            pltpu.SemaphoreType.DMA,
            pltpu.SemaphoreType.DMA,
            pltpu.SemaphoreType.DMA,
        ],
    )


def kernel(inp_feat, W1, b1, W2, b2, Wv1, bv1, W3, b3, W4, b4, Wv2, bv2,
           vox2point_idx):
    ids = vox2point_idx.astype(jnp.int32)
    hist = jnp.zeros((V,), jnp.int32).at[ids].add(1)
    starts = jnp.concatenate(
        [jnp.zeros((1,), jnp.int32), jnp.cumsum(hist, dtype=jnp.int32),
         jnp.full((SPAD - (V + 1),), N, jnp.int32)])
    ids_pad = jnp.concatenate([ids, jnp.full((CH + 32,), V, jnp.int32)])

    pf2 = pl.pallas_call(
        _mlp2_body,
        grid=(N // BLK,),
        in_specs=[
            pl.BlockSpec((BLK, 6), lambda i: (i, 0)),
            _full((6, 64)), _full((1, 64)), _full((64, 64)), _full((1, 64)),
        ],
        out_specs=pl.BlockSpec((BLK, 64), lambda i: (i, 0)),
        out_shape=jax.ShapeDtypeStruct((NPAD, 64), jnp.float32),
    )(inp_feat, W1.T.astype(jnp.bfloat16), b1.reshape(1, 64),
      W2.T.astype(jnp.bfloat16), b2.reshape(1, 64))

    vox1 = _make_segmax(64)(pf2.reshape(-1), ids_pad, starts).reshape(VPAD, 64)

    voxg = pl.pallas_call(
        _voxg_body,
        grid=(VPAD // BLK,),
        in_specs=[
            pl.BlockSpec((BLK, 64), lambda i: (i, 0)),
            _full((64, 64)), _full((1, 64)), _full((64, 128)),
        ],
        out_specs=pl.BlockSpec((BLK, 128), lambda i: (i, 0)),
        out_shape=jax.ShapeDtypeStruct((VPAD, 128), jnp.float32),
    )(vox1, Wv1.T.astype(jnp.bfloat16), bv1.reshape(1, 64),
      W3[:, :64].T.astype(jnp.bfloat16))

    pgf = _make_gather()(voxg, ids)

    pf5 = pl.pallas_call(
        _mid_body,
        grid=(N // BLK,),
        in_specs=[
            pl.BlockSpec((BLK, 64), lambda i: (i, 0)),
            pl.BlockSpec((BLK, 128), lambda i: (i, 0)),
            _full((64, 128)), _full((1, 128)), _full((128, 128)),
            _full((1, 128)),
        ],
        out_specs=pl.BlockSpec((BLK, 128), lambda i: (i, 0)),
        out_shape=jax.ShapeDtypeStruct((NPAD, 128), jnp.float32),
    )(pf2, pgf, W3[:, 64:].T.astype(jnp.bfloat16), b3.reshape(1, 128),
      W4.T.astype(jnp.bfloat16), b4.reshape(1, 128))

    vox2 = _make_segmax(128)(pf5.reshape(-1), ids_pad, starts).reshape(VPAD, 128)

    out = pl.pallas_call(
        _vout_body,
        grid=(10,),
        in_specs=[
            pl.BlockSpec((V // 10, 128), lambda i: (i, 0)),
            _full((128, 128)), _full((1, 128)),
        ],
        out_specs=pl.BlockSpec((V // 10, 128), lambda i: (i, 0)),
        out_shape=jax.ShapeDtypeStruct((V, 128), jnp.float32),
    )(vox2, Wv2.T.astype(jnp.bfloat16), bv2.reshape(1, 128))

    return out


# speedup vs baseline: 1.2612x; 1.0149x over previous
import functools

import jax
import jax.numpy as jnp
from jax import lax
from jax.experimental import pallas as pl
from jax.experimental.pallas import tpu as pltpu
from jax.experimental.pallas import tpu_sc as plsc

N = 640000
V = 10000
NC, NS, L = 2, 16, 16
NW = NC * NS
VPW = 320
VPAD = NW * VPW
SLICE = 336
SPAD = (NW - 1) * VPW + SLICE
CH = 128
NPAD = N + 144
BLK = 8000
BLKV = 2048


def _relu(x):
    return jnp.maximum(x, 0.0)


def _dot(a, b):
    return jnp.dot(a, b, preferred_element_type=jnp.float32)


def _bf(x):
    return x.astype(jnp.bfloat16)


def _mlp2_body(x_ref, w1_ref, b1_ref, w2_ref, b2_ref, o_ref):
    h = _relu(_dot(_bf(x_ref[...]), w1_ref[...]) + b1_ref[...])
    o_ref[...] = _relu(_dot(_bf(h), w2_ref[...]) + b2_ref[...])


def _voxg_body(x_ref, wv_ref, bv_ref, w3l_ref, o_ref):
    h = _relu(_dot(_bf(x_ref[...]), wv_ref[...]) + bv_ref[...])
    o_ref[...] = _dot(_bf(h), w3l_ref[...])


def _mid_body(pf2_ref, pgf_ref, w3r_ref, b3_ref, w4_ref, b4_ref, o_ref):
    pf4 = _relu(pgf_ref[...].astype(jnp.float32)
                + _dot(_bf(pf2_ref[...]), w3r_ref[...]) + b3_ref[...])
    o_ref[...] = _relu(_dot(_bf(pf4), w4_ref[...]) + b4_ref[...])


def _vout_body(x_ref, wv2_ref, bv2_ref, o_ref):
    o_ref[...] = _relu(_dot(_bf(x_ref[...]), wv2_ref[...]) + bv2_ref[...])


def _full(shape):
    return pl.BlockSpec(shape, lambda i: (0, 0))


def _sc_mesh():
    return plsc.VectorSubcoreMesh(
        core_axis_name="c", subcore_axis_name="s",
        num_cores=NC, num_subcores=NS)


def _wid():
    return lax.axis_index("s") * NC + lax.axis_index("c")


def _sread(vec_ref, i):
    return vec_ref[pl.ds(i, L)][0]


def _make_segmax(C):
    VW = 16
    GN = C // VW
    ALIGN = 1
    CHB = CH + ALIGN
    CHW = CHB * C
    IBW = CH + 24

    def body(data_hbm, ids_hbm, starts_hbm, out_hbm,
             starts_v, buf, idb, res, acc, sem0, sem1):
        wid = _wid()
        vlo = wid * VPW
        nv = jnp.minimum(VPW, V - vlo)
        pltpu.sync_copy(starts_hbm.at[pl.ds(vlo, SLICE)], starts_v)

        zeros = tuple(jnp.zeros((VW,), jnp.float32) for _ in range(GN))

        def zero_body(i, c):
            for g in range(GN):
                res[pl.ds(i * C + g * VW, VW)] = zeros[g]
            return c

        lax.fori_loop(0, VPW, zero_body, 0)
        for g in range(GN):
            acc[pl.ds(g * VW, VW)] = zeros[g]

        r0 = _sread(starts_v, 0)
        r1 = _sread(starts_v, nv)
        nch = (r1 - r0 + (CH - 1)) // CH

        def issue(k, b, sem):
            cb = r0 + k * CH
            cba = (cb // ALIGN) * ALIGN
            pltpu.async_copy(
                data_hbm.at[pl.ds(pl.multiple_of(cba * C, C), CHW)],
                buf.at[pl.ds(b * CHW, CHW)], sem)
            pltpu.async_copy(
                ids_hbm.at[pl.ds((cb // 8) * 8, IBW)],
                idb.at[pl.ds(b * IBW, IBW)], sem)

        @pl.when(nch > 0)
        def _():
            issue(0, 0, sem0)

        @pl.when(nch > 1)
        def _():
            issue(1, 1, sem1)

        def step(k, b, sem):
            cb = r0 + k * CH
            boff = b * CHW + (cb - (cb // ALIGN) * ALIGN) * C

            @pl.when(k < nch)
            def _():
                pltpu.make_async_copy(
                    data_hbm.at[pl.ds(0, CHW)],
                    buf.at[pl.ds(b * CHW, CHW)], sem).wait()
                pltpu.make_async_copy(
                    ids_hbm.at[pl.ds(0, IBW)],
                    idb.at[pl.ds(b * IBW, IBW)], sem).wait()

            ce = jnp.minimum(cb + CH, r1)
            rem = jnp.maximum(ce - cb, 0)
            ioff = b * IBW + (cb - (cb // 8) * 8)

            def row_step(r, id_r, accs):
                id_n = idb[pl.ds(ioff + r + 1, L)][0]
                cur = tuple(
                    jnp.maximum(accs[g], buf[pl.ds(boff + r * C + g * VW, VW)])
                    for g in range(GN))
                done = id_n != id_r

                @pl.when(done)
                def _():
                    for g in range(GN):
                        res[pl.ds((id_r - vlo) * C + g * VW, VW)] = cur[g]

                keep = (~done).astype(jnp.float32)
                return id_n, tuple(cur[g] * keep for g in range(GN))

            def rb2(t, st):
                id_r, accs = st
                id_r, accs = row_step(2 * t, id_r, accs)
                id_r, accs = row_step(2 * t + 1, id_r, accs)
                return id_r, accs

            def rb1(r, st):
                id_r, accs = st
                return row_step(r, id_r, accs)

            st = (idb[pl.ds(ioff, L)][0],
                  tuple(acc[pl.ds(g * VW, VW)] for g in range(GN)))
            st = lax.fori_loop(0, rem // 2, rb2, st)
            st = lax.fori_loop(2 * (rem // 2), rem, rb1, st)
            for g in range(GN):
                acc[pl.ds(g * VW, VW)] = st[1][g]

            @pl.when(k + 2 < nch)
            def _():
                issue(k + 2, b, sem)

        def pair(t, c):
            step(2 * t, 0, sem0)
            step(2 * t + 1, 1, sem1)
            return c

        lax.fori_loop(0, (nch + 1) // 2, pair, 0)
        pltpu.sync_copy(
            res, out_hbm.at[pl.ds(pl.multiple_of(vlo * C, C), VPW * C)])

    return pl.kernel(
        body,
        out_type=jax.ShapeDtypeStruct((VPAD * C,), jnp.float32),
        mesh=_sc_mesh(),
        scratch_types=[
            pltpu.VMEM((SLICE,), jnp.int32),
            pltpu.VMEM((2 * CHW,), jnp.float32),
            pltpu.VMEM((2 * IBW,), jnp.int32),
            pltpu.VMEM((VPW * C,), jnp.float32),
            pltpu.VMEM((C,), jnp.float32),
            pltpu.SemaphoreType.DMA,
            pltpu.SemaphoreType.DMA,
        ],
    )


_G = 80
_RPW = N // NW
_BB = 400
_NG = _BB // _G
_NBB = _RPW // _BB


def _make_gather():
    def body(table_hbm, ids_hbm, out_hbm, idx_v, rows_v,
             sg0, sg1, ss0, ss1):
        r0w = _wid() * _RPW
        pltpu.sync_copy(ids_hbm.at[pl.ds(r0w, _RPW)], idx_v)

        def issue_gathers(k, b, sem):
            for g in range(_NG):
                pltpu.async_copy(
                    table_hbm.at[idx_v.at[pl.ds(k * _BB + g * _G, _G)]],
                    rows_v.at[b].at[pl.ds(g * _G, _G)], sem)

        issue_gathers(0, 0, sg0)
        issue_gathers(1, 1, sg1)

        def store_desc(k, b, sem):
            return pltpu.make_async_copy(
                rows_v.at[b], out_hbm.at[pl.ds(r0w + k * _BB, _BB)], sem)

        def step(k, b, sg, ss):
            pltpu.make_async_copy(
                out_hbm.at[pl.ds(r0w, _BB)], rows_v.at[b], sg).wait()
            store_desc(k, b, ss).start()

            @pl.when(k + 2 < _NBB)
            def _():
                store_desc(k, b, ss).wait()
                issue_gathers(k + 2, b, sg)

        def pair(t, c):
            step(2 * t, 0, sg0, ss0)
            step(2 * t + 1, 1, sg1, ss1)
            return c

        lax.fori_loop(0, _NBB // 2, pair, 0)
        store_desc(_NBB - 2, 0, ss0).wait()
        store_desc(_NBB - 1, 1, ss1).wait()

    return pl.kernel(
        body,
        out_type=jax.ShapeDtypeStruct((N, 128), jnp.float32),
        mesh=_sc_mesh(),
        scratch_types=[
            pltpu.VMEM((_RPW,), jnp.int32),
            pltpu.VMEM((2, _BB, 128), jnp.float32),
            pltpu.SemaphoreType.DMA,
            pltpu.SemaphoreType.DMA,
            pltpu.SemaphoreType.DMA,
            pltpu.SemaphoreType.DMA,
        ],
    )


def kernel(inp_feat, W1, b1, W2, b2, Wv1, bv1, W3, b3, W4, b4, Wv2, bv2,
           vox2point_idx):
    ids = vox2point_idx.astype(jnp.int32)
    hist = jnp.zeros((V,), jnp.int32).at[ids].add(1)
    starts = jnp.concatenate(
        [jnp.zeros((1,), jnp.int32), jnp.cumsum(hist, dtype=jnp.int32),
         jnp.full((SPAD - (V + 1),), N, jnp.int32)])
    ids_pad = jnp.concatenate([ids, jnp.full((CH + 32,), V, jnp.int32)])

    pf2 = pl.pallas_call(
        _mlp2_body,
        grid=(N // BLK,),
        in_specs=[
            pl.BlockSpec((BLK, 6), lambda i: (i, 0)),
            _full((6, 64)), _full((1, 64)), _full((64, 64)), _full((1, 64)),
        ],
        out_specs=pl.BlockSpec((BLK, 64), lambda i: (i, 0)),
        out_shape=jax.ShapeDtypeStruct((NPAD, 64), jnp.float32),
    )(inp_feat, W1.T.astype(jnp.bfloat16), b1.reshape(1, 64),
      W2.T.astype(jnp.bfloat16), b2.reshape(1, 64))

    vox1 = _make_segmax(64)(pf2.reshape(-1), ids_pad, starts).reshape(VPAD, 64)

    voxg = pl.pallas_call(
        _voxg_body,
        grid=(VPAD // BLKV,),
        in_specs=[
            pl.BlockSpec((BLKV, 64), lambda i: (i, 0)),
            _full((64, 64)), _full((1, 64)), _full((64, 128)),
        ],
        out_specs=pl.BlockSpec((BLKV, 128), lambda i: (i, 0)),
        out_shape=jax.ShapeDtypeStruct((VPAD, 128), jnp.float32),
    )(vox1, Wv1.T.astype(jnp.bfloat16), bv1.reshape(1, 64),
      W3[:, :64].T.astype(jnp.bfloat16))

    pgf = _make_gather()(voxg, ids)

    pf5 = pl.pallas_call(
        _mid_body,
        grid=(N // BLK,),
        in_specs=[
            pl.BlockSpec((BLK, 64), lambda i: (i, 0)),
            pl.BlockSpec((BLK, 128), lambda i: (i, 0)),
            _full((64, 128)), _full((1, 128)), _full((128, 128)),
            _full((1, 128)),
        ],
        out_specs=pl.BlockSpec((BLK, 128), lambda i: (i, 0)),
        out_shape=jax.ShapeDtypeStruct((NPAD, 128), jnp.float32),
    )(pf2, pgf, W3[:, 64:].T.astype(jnp.bfloat16), b3.reshape(1, 128),
      W4.T.astype(jnp.bfloat16), b4.reshape(1, 128))

    vox2 = _make_segmax(128)(pf5.reshape(-1), ids_pad, starts).reshape(VPAD, 128)

    out = pl.pallas_call(
        _vout_body,
        grid=(10,),
        in_specs=[
            pl.BlockSpec((V // 10, 128), lambda i: (i, 0)),
            _full((128, 128)), _full((1, 128)),
        ],
        out_specs=pl.BlockSpec((V // 10, 128), lambda i: (i, 0)),
        out_shape=jax.ShapeDtypeStruct((V, 128), jnp.float32),
    )(vox2, Wv2.T.astype(jnp.bfloat16), bv2.reshape(1, 128))

    return out


# speedup vs baseline: 1.2627x; 1.0011x over previous
import functools

import jax
import jax.numpy as jnp
from jax import lax
from jax.experimental import pallas as pl
from jax.experimental.pallas import tpu as pltpu
from jax.experimental.pallas import tpu_sc as plsc

N = 640000
V = 10000
NC, NS, L = 2, 16, 16
NW = NC * NS
VPW = 320
VPAD = NW * VPW
SLICE = 336
SPAD = (NW - 1) * VPW + SLICE
CH = 256
NPAD = N + 144
BLK = 8000
BLKV = 2048


def _relu(x):
    return jnp.maximum(x, 0.0)


def _dot(a, b):
    return jnp.dot(a, b, preferred_element_type=jnp.float32)


def _bf(x):
    return x.astype(jnp.bfloat16)


def _mlp2_body(x_ref, w1_ref, b1_ref, w2_ref, b2_ref, o_ref):
    h = _relu(_dot(_bf(x_ref[...]), w1_ref[...]) + b1_ref[...])
    o_ref[...] = _relu(_dot(_bf(h), w2_ref[...]) + b2_ref[...])


def _voxg_body(x_ref, wv_ref, bv_ref, w3l_ref, o_ref):
    h = _relu(_dot(_bf(x_ref[...]), wv_ref[...]) + bv_ref[...])
    o_ref[...] = _dot(_bf(h), w3l_ref[...])


def _mid_body(pf2_ref, pgf_ref, w3r_ref, b3_ref, w4_ref, b4_ref, o_ref):
    pf4 = _relu(pgf_ref[...].astype(jnp.float32)
                + _dot(_bf(pf2_ref[...]), w3r_ref[...]) + b3_ref[...])
    o_ref[...] = _relu(_dot(_bf(pf4), w4_ref[...]) + b4_ref[...])


def _vout_body(x_ref, wv2_ref, bv2_ref, o_ref):
    o_ref[...] = _relu(_dot(_bf(x_ref[...]), wv2_ref[...]) + bv2_ref[...])


def _full(shape):
    return pl.BlockSpec(shape, lambda i: (0, 0))


def _sc_mesh():
    return plsc.VectorSubcoreMesh(
        core_axis_name="c", subcore_axis_name="s",
        num_cores=NC, num_subcores=NS)


def _wid():
    return lax.axis_index("s") * NC + lax.axis_index("c")


def _sread(vec_ref, i):
    return vec_ref[pl.ds(i, L)][0]


def _make_segmax(C):
    VW = 16
    GN = C // VW
    ALIGN = 1
    CHB = CH + ALIGN
    CHW = CHB * C
    IBW = CH + 24

    def body(data_hbm, ids_hbm, starts_hbm, out_hbm,
             starts_v, buf, idb, res, acc, sem0, sem1):
        wid = _wid()
        vlo = wid * VPW
        nv = jnp.minimum(VPW, V - vlo)
        pltpu.sync_copy(starts_hbm.at[pl.ds(vlo, SLICE)], starts_v)

        zeros = tuple(jnp.zeros((VW,), jnp.float32) for _ in range(GN))

        def zero_body(i, c):
            for g in range(GN):
                res[pl.ds(i * C + g * VW, VW)] = zeros[g]
            return c

        lax.fori_loop(0, VPW, zero_body, 0)
        for g in range(GN):
            acc[pl.ds(g * VW, VW)] = zeros[g]

        r0 = _sread(starts_v, 0)
        r1 = _sread(starts_v, nv)
        nch = (r1 - r0 + (CH - 1)) // CH

        def issue(k, b, sem):
            cb = r0 + k * CH
            cba = (cb // ALIGN) * ALIGN
            pltpu.async_copy(
                data_hbm.at[pl.ds(pl.multiple_of(cba * C, C), CHW)],
                buf.at[pl.ds(b * CHW, CHW)], sem)
            pltpu.async_copy(
                ids_hbm.at[pl.ds((cb // 8) * 8, IBW)],
                idb.at[pl.ds(b * IBW, IBW)], sem)

        @pl.when(nch > 0)
        def _():
            issue(0, 0, sem0)

        @pl.when(nch > 1)
        def _():
            issue(1, 1, sem1)

        def step(k, b, sem):
            cb = r0 + k * CH
            boff = b * CHW + (cb - (cb // ALIGN) * ALIGN) * C

            @pl.when(k < nch)
            def _():
                pltpu.make_async_copy(
                    data_hbm.at[pl.ds(0, CHW)],
                    buf.at[pl.ds(b * CHW, CHW)], sem).wait()
                pltpu.make_async_copy(
                    ids_hbm.at[pl.ds(0, IBW)],
                    idb.at[pl.ds(b * IBW, IBW)], sem).wait()

            ce = jnp.minimum(cb + CH, r1)
            rem = jnp.maximum(ce - cb, 0)
            ioff = b * IBW + (cb - (cb // 8) * 8)

            def row_step(r, id_r, accs):
                id_n = idb[pl.ds(ioff + r + 1, L)][0]
                cur = tuple(
                    jnp.maximum(accs[g], buf[pl.ds(boff + r * C + g * VW, VW)])
                    for g in range(GN))
                done = id_n != id_r

                @pl.when(done)
                def _():
                    for g in range(GN):
                        res[pl.ds((id_r - vlo) * C + g * VW, VW)] = cur[g]

                keep = (~done).astype(jnp.float32)
                return id_n, tuple(cur[g] * keep for g in range(GN))

            def rb2(t, st):
                id_r, accs = st
                id_r, accs = row_step(2 * t, id_r, accs)
                id_r, accs = row_step(2 * t + 1, id_r, accs)
                return id_r, accs

            def rb1(r, st):
                id_r, accs = st
                return row_step(r, id_r, accs)

            st = (idb[pl.ds(ioff, L)][0],
                  tuple(acc[pl.ds(g * VW, VW)] for g in range(GN)))
            st = lax.fori_loop(0, rem // 2, rb2, st)
            st = lax.fori_loop(2 * (rem // 2), rem, rb1, st)
            for g in range(GN):
                acc[pl.ds(g * VW, VW)] = st[1][g]

            @pl.when(k + 2 < nch)
            def _():
                issue(k + 2, b, sem)

        def pair(t, c):
            step(2 * t, 0, sem0)
            step(2 * t + 1, 1, sem1)
            return c

        lax.fori_loop(0, (nch + 1) // 2, pair, 0)
        pltpu.sync_copy(
            res, out_hbm.at[pl.ds(pl.multiple_of(vlo * C, C), VPW * C)])

    return pl.kernel(
        body,
        out_type=jax.ShapeDtypeStruct((VPAD * C,), jnp.float32),
        mesh=_sc_mesh(),
        scratch_types=[
            pltpu.VMEM((SLICE,), jnp.int32),
            pltpu.VMEM((2 * CHW,), jnp.float32),
            pltpu.VMEM((2 * IBW,), jnp.int32),
            pltpu.VMEM((VPW * C,), jnp.float32),
            pltpu.VMEM((C,), jnp.float32),
            pltpu.SemaphoreType.DMA,
            pltpu.SemaphoreType.DMA,
        ],
    )


_G = 80
_RPW = N // NW
_BB = 400
_NG = _BB // _G
_NBB = _RPW // _BB


def _make_gather():
    def body(table_hbm, ids_hbm, out_hbm, idx_v, rows_v,
             sg0, sg1, ss0, ss1):
        r0w = _wid() * _RPW
        pltpu.sync_copy(ids_hbm.at[pl.ds(r0w, _RPW)], idx_v)

        def issue_gathers(k, b, sem):
            for g in range(_NG):
                pltpu.async_copy(
                    table_hbm.at[idx_v.at[pl.ds(k * _BB + g * _G, _G)]],
                    rows_v.at[b].at[pl.ds(g * _G, _G)], sem)

        issue_gathers(0, 0, sg0)
        issue_gathers(1, 1, sg1)

        def store_desc(k, b, sem):
            return pltpu.make_async_copy(
                rows_v.at[b], out_hbm.at[pl.ds(r0w + k * _BB, _BB)], sem)

        def step(k, b, sg, ss):
            pltpu.make_async_copy(
                out_hbm.at[pl.ds(r0w, _BB)], rows_v.at[b], sg).wait()
            store_desc(k, b, ss).start()

            @pl.when(k + 2 < _NBB)
            def _():
                store_desc(k, b, ss).wait()
                issue_gathers(k + 2, b, sg)

        def pair(t, c):
            step(2 * t, 0, sg0, ss0)
            step(2 * t + 1, 1, sg1, ss1)
            return c

        lax.fori_loop(0, _NBB // 2, pair, 0)
        store_desc(_NBB - 2, 0, ss0).wait()
        store_desc(_NBB - 1, 1, ss1).wait()

    return pl.kernel(
        body,
        out_type=jax.ShapeDtypeStruct((N, 128), jnp.float32),
        mesh=_sc_mesh(),
        scratch_types=[
            pltpu.VMEM((_RPW,), jnp.int32),
            pltpu.VMEM((2, _BB, 128), jnp.float32),
            pltpu.SemaphoreType.DMA,
            pltpu.SemaphoreType.DMA,
            pltpu.SemaphoreType.DMA,
            pltpu.SemaphoreType.DMA,
        ],
    )


def kernel(inp_feat, W1, b1, W2, b2, Wv1, bv1, W3, b3, W4, b4, Wv2, bv2,
           vox2point_idx):
    ids = vox2point_idx.astype(jnp.int32)
    hist = jnp.zeros((V,), jnp.int32).at[ids].add(1)
    starts = jnp.concatenate(
        [jnp.zeros((1,), jnp.int32), jnp.cumsum(hist, dtype=jnp.int32),
         jnp.full((SPAD - (V + 1),), N, jnp.int32)])
    ids_pad = jnp.concatenate([ids, jnp.full((CH + 32,), V, jnp.int32)])

    pf2 = pl.pallas_call(
        _mlp2_body,
        grid=(N // BLK,),
        in_specs=[
            pl.BlockSpec((BLK, 6), lambda i: (i, 0)),
            _full((6, 64)), _full((1, 64)), _full((64, 64)), _full((1, 64)),
        ],
        out_specs=pl.BlockSpec((BLK, 64), lambda i: (i, 0)),
        out_shape=jax.ShapeDtypeStruct((NPAD, 64), jnp.float32),
    )(inp_feat, W1.T.astype(jnp.bfloat16), b1.reshape(1, 64),
      W2.T.astype(jnp.bfloat16), b2.reshape(1, 64))

    vox1 = _make_segmax(64)(pf2.reshape(-1), ids_pad, starts).reshape(VPAD, 64)

    voxg = pl.pallas_call(
        _voxg_body,
        grid=(VPAD // BLKV,),
        in_specs=[
            pl.BlockSpec((BLKV, 64), lambda i: (i, 0)),
            _full((64, 64)), _full((1, 64)), _full((64, 128)),
        ],
        out_specs=pl.BlockSpec((BLKV, 128), lambda i: (i, 0)),
        out_shape=jax.ShapeDtypeStruct((VPAD, 128), jnp.float32),
    )(vox1, Wv1.T.astype(jnp.bfloat16), bv1.reshape(1, 64),
      W3[:, :64].T.astype(jnp.bfloat16))

    pgf = _make_gather()(voxg, ids)

    pf5 = pl.pallas_call(
        _mid_body,
        grid=(N // BLK,),
        in_specs=[
            pl.BlockSpec((BLK, 64), lambda i: (i, 0)),
            pl.BlockSpec((BLK, 128), lambda i: (i, 0)),
            _full((64, 128)), _full((1, 128)), _full((128, 128)),
            _full((1, 128)),
        ],
        out_specs=pl.BlockSpec((BLK, 128), lambda i: (i, 0)),
        out_shape=jax.ShapeDtypeStruct((NPAD, 128), jnp.float32),
    )(pf2, pgf, W3[:, 64:].T.astype(jnp.bfloat16), b3.reshape(1, 128),
      W4.T.astype(jnp.bfloat16), b4.reshape(1, 128))

    vox2 = _make_segmax(128)(pf5.reshape(-1), ids_pad, starts).reshape(VPAD, 128)

    out = pl.pallas_call(
        _vout_body,
        grid=(10,),
        in_specs=[
            pl.BlockSpec((V // 10, 128), lambda i: (i, 0)),
            _full((128, 128)), _full((1, 128)),
        ],
        out_specs=pl.BlockSpec((V // 10, 128), lambda i: (i, 0)),
        out_shape=jax.ShapeDtypeStruct((V, 128), jnp.float32),
    )(vox2, Wv2.T.astype(jnp.bfloat16), bv2.reshape(1, 128))

    return out
